# baseline (speedup 1.0000x reference)
import functools

import jax
import jax.numpy as jnp
from jax import lax
from jax.experimental import pallas as pl
from jax.experimental.pallas import tpu as pltpu
from jax.experimental.pallas import tpu_sc as plsc

V = 16384
N = 4096
VB = 1024
GRID = V // VB

_SC_INFO = plsc.get_sparse_core_info()
_NC = _SC_INFO.num_cores
_NS = _SC_INFO.num_subcores
_NW = _NC * _NS
_BPW = V // _NW
_LANES = _SC_INFO.num_lanes


def _nn_body(af_ref, bf_ref, o_ref):
    ab = jnp.dot(af_ref[...], bf_ref[...], preferred_element_type=jnp.float32)

    af = af_ref[...]
    a2 = jnp.sum(af * af, axis=1, keepdims=True)
    bf = bf_ref[...]
    b2 = jnp.sum(bf * bf, axis=0, keepdims=True)

    d2 = jnp.maximum(a2 + b2 - 2.0 * ab, 0.0)
    v = jnp.sqrt(d2)
    minval = jnp.min(v, axis=1, keepdims=True)
    iota = jax.lax.broadcasted_iota(jnp.int32, v.shape, 1)
    idx = jnp.min(jnp.where(v == minval, iota, jnp.int32(N)),
                  axis=1)
    o_ref[...] = idx.reshape(1, 1, VB)


def _sc_gather_body(idx_hbm, f0_hbm, f1_hbm, out_hbm,
                    idx_v, g0_v, g1_v, out_v, sem):
    wid = lax.axis_index("s") * _NC + lax.axis_index("c")
    base = wid * _BPW
    pltpu.sync_copy(idx_hbm.at[pl.ds(base, _BPW)], idx_v)
    pltpu.async_copy(f0_hbm.at[idx_v], g0_v, sem).wait()
    pltpu.async_copy(f1_hbm.at[idx_v], g1_v, sem).wait()
    for i in range(_BPW // _LANES):
        sl = pl.ds(i * _LANES, _LANES)
        s = 0.6 * jnp.abs(g0_v[sl]) + 0.4 * g1_v[sl]
        out_v[sl] = 1.0 / (1.0 + jnp.exp(-s))
    pltpu.sync_copy(out_v, out_hbm.at[pl.ds(base, _BPW)])


def kernel(radar_voxel_feat, lidar_voxel_coords, radar_voxel_coords):
    af = lidar_voxel_coords.astype(jnp.float32)
    bf = radar_voxel_coords.astype(jnp.float32).T

    idx = pl.pallas_call(
        _nn_body,
        grid=(GRID,),
        in_specs=[
            pl.BlockSpec((VB, 3), lambda i: (i, 0)),
            pl.BlockSpec((3, N), lambda i: (0, 0)),
        ],
        out_specs=pl.BlockSpec((1, 1, VB), lambda i: (i, 0, 0)),
        out_shape=jax.ShapeDtypeStruct((GRID, 1, VB), jnp.int32),
    )(af, bf).reshape(V)

    f0 = radar_voxel_feat[:, 0]
    f1 = radar_voxel_feat[:, 1]

    mesh = plsc.VectorSubcoreMesh(core_axis_name="c", subcore_axis_name="s")
    gather = functools.partial(
        pl.kernel,
        mesh=mesh,
        out_type=jax.ShapeDtypeStruct((V,), jnp.float32),
        scratch_types=[
            pltpu.VMEM((_BPW,), jnp.int32),
            pltpu.VMEM((_BPW,), jnp.float32),
            pltpu.VMEM((_BPW,), jnp.float32),
            pltpu.VMEM((_BPW,), jnp.float32),
            pltpu.SemaphoreType.DMA,
        ],
    )(_sc_gather_body)
    return gather(idx, f0, f1)
